# baseline (speedup 1.0000x reference)
import functools

import jax
import jax.numpy as jnp
from jax.experimental import pallas as pl

_K = 8


def _sample_concrete_block(logits_ref, uniform_ref, out_ref):
    eps = jnp.finfo(jnp.float32).eps
    D = logits_ref.shape[-1]
    L = logits_ref[:]
    Lmax = jnp.max(L, axis=-1, keepdims=True)
    expL = jnp.exp(2.0 * (L - Lmax))
    acc = jnp.full_like(L, -jnp.inf)
    for k in range(_K):
        u = jnp.clip(uniform_ref[:, k * D:(k + 1) * D], eps, 1.0)
        rw = 1.0 / jnp.log2(u)
        e = expL * (rw * rw)
        s = jnp.sum(e, axis=-1, keepdims=True)
        acc = jnp.maximum(acc, e * (1.0 / s))
    out_ref[:] = acc


@functools.partial(jax.jit, static_argnames=("interpret",))
def kernel(logits, uniform, interpret=False):
    B, D = logits.shape
    _, K, _ = uniform.shape
    uniform2 = uniform.reshape(B, K * D)
    BB = 8
    return pl.pallas_call(
        _sample_concrete_block,
        grid=(B // BB,),
        in_specs=[
            pl.BlockSpec((BB, D), lambda b: (b, 0)),
            pl.BlockSpec((BB, K * D), lambda b: (b, 0)),
        ],
        out_specs=pl.BlockSpec((BB, D), lambda b: (b, 0)),
        out_shape=jax.ShapeDtypeStruct((B, D), jnp.float32),
        interpret=interpret,
    )(logits, uniform2)


# speedup vs baseline: 1.6390x; 1.6390x over previous
import functools

import jax
import jax.numpy as jnp
from jax.experimental import pallas as pl
from jax.experimental.pallas import tpu as pltpu

_K = 8
_BB = 8
_NBUF = 4


def _sample_concrete_step(logits_ref, uniform_hbm, out_ref, ubuf, expl_ref,
                          sems, *, nsteps):
    g = pl.program_id(0)
    k = g % _K

    def start_dma(step):
        b = step // _K
        kk = step % _K
        pltpu.make_async_copy(
            uniform_hbm.at[pl.ds(b * _BB, _BB), kk],
            ubuf.at[step % _NBUF],
            sems.at[step % _NBUF],
        ).start()

    @pl.when(g == 0)
    def _prologue():
        for i in range(_NBUF):
            start_dma(i)

    pltpu.make_async_copy(
        uniform_hbm.at[pl.ds(0, _BB), 0],
        ubuf.at[g % _NBUF],
        sems.at[g % _NBUF],
    ).wait()

    @pl.when(k == 0)
    def _compute_expl():
        L = logits_ref[:]
        Lmax = jnp.max(L, axis=-1, keepdims=True)
        expl_ref[:] = jnp.exp(2.0 * (L - Lmax))

    eps = jnp.finfo(jnp.float32).eps
    u = jnp.clip(ubuf[g % _NBUF], eps, 1.0)
    rw = 1.0 / jnp.log2(u)
    e = expl_ref[:] * (rw * rw)
    s = jnp.sum(e, axis=-1, keepdims=True)
    v = e * (1.0 / s)

    @pl.when(k == 0)
    def _init_out():
        out_ref[:] = v

    @pl.when(k != 0)
    def _acc_out():
        out_ref[:] = jnp.maximum(out_ref[:], v)

    @pl.when(g + _NBUF < nsteps)
    def _next_dma():
        start_dma(g + _NBUF)


@functools.partial(jax.jit, static_argnames=("interpret",))
def kernel(logits, uniform, interpret=False):
    B, D = logits.shape
    _, K, _ = uniform.shape
    nsteps = (B // _BB) * _K
    return pl.pallas_call(
        functools.partial(_sample_concrete_step, nsteps=nsteps),
        grid=(nsteps,),
        in_specs=[
            pl.BlockSpec((_BB, D), lambda g: (g // _K, 0)),
            pl.BlockSpec(memory_space=pl.ANY),
        ],
        out_specs=pl.BlockSpec((_BB, D), lambda g: (g // _K, 0)),
        out_shape=jax.ShapeDtypeStruct((B, D), jnp.float32),
        scratch_shapes=[
            pltpu.VMEM((_NBUF, _BB, D), jnp.float32),
            pltpu.VMEM((_BB, D), jnp.float32),
            pltpu.SemaphoreType.DMA((_NBUF,)),
        ],
        interpret=interpret,
    )(logits, uniform)


# speedup vs baseline: 2.2423x; 1.3681x over previous
import functools

import jax
import jax.numpy as jnp
from jax.experimental import pallas as pl
from jax.experimental.pallas import tpu as pltpu

_K = 8
_BB = 8
_NBUF = 8


def _sample_concrete_block(logits_ref, uniform_hbm, out_ref, ubuf, sems, *,
                           nsteps):
    nb = pl.program_id(0)

    def start_dma(step):
        b = step // _K
        kk = step % _K
        pltpu.make_async_copy(
            uniform_hbm.at[pl.ds(b * _BB, _BB), kk],
            ubuf.at[step % _NBUF],
            sems.at[step % _NBUF],
        ).start()

    @pl.when(nb == 0)
    def _prologue():
        for i in range(_NBUF):
            start_dma(i)

    L = logits_ref[:]
    Lmax = jnp.max(L, axis=-1, keepdims=True)
    expL = jnp.exp(2.0 * (L - Lmax))
    eps = jnp.finfo(jnp.float32).eps

    acc = None
    for k in range(_K):
        g = nb * _K + k
        pltpu.make_async_copy(
            uniform_hbm.at[pl.ds(0, _BB), 0],
            ubuf.at[g % _NBUF],
            sems.at[g % _NBUF],
        ).wait()
        u = jnp.clip(ubuf[g % _NBUF], eps, 1.0)
        rw = 1.0 / jnp.log2(u)
        e = expL * (rw * rw)
        s = jnp.sum(e, axis=-1, keepdims=True)
        v = e * (1.0 / s)
        acc = v if k == 0 else jnp.maximum(acc, v)

        @pl.when(g + _NBUF < nsteps)
        def _next_dma():
            start_dma(g + _NBUF)

    out_ref[:] = acc


@functools.partial(jax.jit, static_argnames=("interpret",))
def kernel(logits, uniform, interpret=False):
    B, D = logits.shape
    _, K, _ = uniform.shape
    nblocks = B // _BB
    nsteps = nblocks * _K
    return pl.pallas_call(
        functools.partial(_sample_concrete_block, nsteps=nsteps),
        grid=(nblocks,),
        in_specs=[
            pl.BlockSpec((_BB, D), lambda b: (b, 0)),
            pl.BlockSpec(memory_space=pl.ANY),
        ],
        out_specs=pl.BlockSpec((_BB, D), lambda b: (b, 0)),
        out_shape=jax.ShapeDtypeStruct((B, D), jnp.float32),
        scratch_shapes=[
            pltpu.VMEM((_NBUF, _BB, D), jnp.float32),
            pltpu.SemaphoreType.DMA((_NBUF,)),
        ],
        interpret=interpret,
    )(logits, uniform)


# speedup vs baseline: 2.7347x; 1.2196x over previous
import functools

import jax
import jax.numpy as jnp
from jax.experimental import pallas as pl
from jax.experimental.pallas import tpu as pltpu

_K = 8
_BB = 16
_NBUF = 8


def _sample_concrete_block(logits_ref, uniform_hbm, out_ref, ubuf, sems, *,
                           nsteps):
    nb = pl.program_id(0)

    def start_dma(step):
        b = step // _K
        kk = step % _K
        pltpu.make_async_copy(
            uniform_hbm.at[pl.ds(b * _BB, _BB), kk],
            ubuf.at[step % _NBUF],
            sems.at[step % _NBUF],
        ).start()

    @pl.when(nb == 0)
    def _prologue():
        for i in range(_NBUF):
            start_dma(i)

    L = logits_ref[:]
    Lmax = jnp.max(L, axis=-1, keepdims=True)
    expL = jnp.exp(2.0 * (L - Lmax))
    eps = jnp.finfo(jnp.float32).eps

    acc = None
    for k in range(_K):
        g = nb * _K + k
        pltpu.make_async_copy(
            uniform_hbm.at[pl.ds(0, _BB), 0],
            ubuf.at[g % _NBUF],
            sems.at[g % _NBUF],
        ).wait()
        u = jnp.clip(ubuf[g % _NBUF], eps, 1.0)
        rw = 1.0 / jnp.log2(u)
        e = expL * (rw * rw)
        s = jnp.sum(e, axis=-1, keepdims=True)
        v = e * (1.0 / s)
        acc = v if k == 0 else jnp.maximum(acc, v)

        @pl.when(g + _NBUF < nsteps)
        def _next_dma():
            start_dma(g + _NBUF)

    out_ref[:] = acc


@functools.partial(jax.jit, static_argnames=("interpret",))
def kernel(logits, uniform, interpret=False):
    B, D = logits.shape
    _, K, _ = uniform.shape
    nblocks = B // _BB
    nsteps = nblocks * _K
    return pl.pallas_call(
        functools.partial(_sample_concrete_block, nsteps=nsteps),
        grid=(nblocks,),
        in_specs=[
            pl.BlockSpec((_BB, D), lambda b: (b, 0)),
            pl.BlockSpec(memory_space=pl.ANY),
        ],
        out_specs=pl.BlockSpec((_BB, D), lambda b: (b, 0)),
        out_shape=jax.ShapeDtypeStruct((B, D), jnp.float32),
        scratch_shapes=[
            pltpu.VMEM((_NBUF, _BB, D), jnp.float32),
            pltpu.SemaphoreType.DMA((_NBUF,)),
        ],
        interpret=interpret,
    )(logits, uniform)


# speedup vs baseline: 2.7954x; 1.0222x over previous
import functools

import jax
import jax.numpy as jnp
from jax.experimental import pallas as pl
from jax.experimental.pallas import tpu as pltpu

_K = 8
_BB = 32
_NBUF = 4


def _sample_concrete_block(logits_ref, uniform_hbm, out_ref, ubuf, sems, *,
                           nsteps):
    nb = pl.program_id(0)

    def start_dma(step):
        b = step // _K
        kk = step % _K
        pltpu.make_async_copy(
            uniform_hbm.at[pl.ds(b * _BB, _BB), kk],
            ubuf.at[step % _NBUF],
            sems.at[step % _NBUF],
        ).start()

    @pl.when(nb == 0)
    def _prologue():
        for i in range(_NBUF):
            start_dma(i)

    L = logits_ref[:]
    Lmax = jnp.max(L, axis=-1, keepdims=True)
    expL = jnp.exp(2.0 * (L - Lmax))
    eps = jnp.finfo(jnp.float32).eps

    acc = None
    for k in range(_K):
        g = nb * _K + k
        pltpu.make_async_copy(
            uniform_hbm.at[pl.ds(0, _BB), 0],
            ubuf.at[g % _NBUF],
            sems.at[g % _NBUF],
        ).wait()
        u = jnp.clip(ubuf[g % _NBUF], eps, 1.0)
        rw = 1.0 / jnp.log2(u)
        e = expL * (rw * rw)
        s = jnp.sum(e, axis=-1, keepdims=True)
        v = e * (1.0 / s)
        acc = v if k == 0 else jnp.maximum(acc, v)

        @pl.when(g + _NBUF < nsteps)
        def _next_dma():
            start_dma(g + _NBUF)

    out_ref[:] = acc


@functools.partial(jax.jit, static_argnames=("interpret",))
def kernel(logits, uniform, interpret=False):
    B, D = logits.shape
    _, K, _ = uniform.shape
    nblocks = B // _BB
    nsteps = nblocks * _K
    return pl.pallas_call(
        functools.partial(_sample_concrete_block, nsteps=nsteps),
        grid=(nblocks,),
        in_specs=[
            pl.BlockSpec((_BB, D), lambda b: (b, 0)),
            pl.BlockSpec(memory_space=pl.ANY),
        ],
        out_specs=pl.BlockSpec((_BB, D), lambda b: (b, 0)),
        out_shape=jax.ShapeDtypeStruct((B, D), jnp.float32),
        scratch_shapes=[
            pltpu.VMEM((_NBUF, _BB, D), jnp.float32),
            pltpu.SemaphoreType.DMA((_NBUF,)),
        ],
        interpret=interpret,
    )(logits, uniform)
